# Initial kernel scaffold; baseline (speedup 1.0000x reference)
import jax
import jax.numpy as jnp
from jax.experimental import pallas as pl


def kernel(x, attribute_feat):
    raise NotImplementedError("write your pallas kernel here")


# speedup vs baseline: 1.9249x; 1.9249x over previous
import jax
import jax.numpy as jnp
from jax.experimental import pallas as pl


_EPS = 1e-12


def _body(x_ref, a_ref, o_ref):
    xb = x_ref[...]
    a = a_ref[...]
    xnsq = jnp.sum(xb * xb, axis=1, keepdims=True)
    xn = xb / jnp.maximum(jnp.sqrt(xnsq), _EPS)
    dots = jnp.sum(a * xn[None, :, :], axis=2)
    nsq = jnp.sum(a * a, axis=2)
    na = jnp.maximum(jnp.sqrt(nsq), _EPS)
    scores = dots / na
    m = jnp.max(scores, axis=0, keepdims=True)
    e = jnp.exp(scores - m)
    w = e / jnp.sum(e, axis=0, keepdims=True)
    coef = (w / na)[:, :, None]
    o_ref[...] = jnp.sum(a * coef, axis=0)


def kernel(x, attribute_feat):
    B, d = x.shape
    A = attribute_feat.shape[0]
    R = 256
    return pl.pallas_call(
        _body,
        grid=(B // R,),
        in_specs=[
            pl.BlockSpec((R, d), lambda i: (i, 0)),
            pl.BlockSpec((A, R, d), lambda i: (0, i, 0)),
        ],
        out_specs=pl.BlockSpec((R, d), lambda i: (i, 0)),
        out_shape=jax.ShapeDtypeStruct((B, d), jnp.float32),
    )(x, attribute_feat)


# speedup vs baseline: 2.8529x; 1.4821x over previous
import jax
import jax.numpy as jnp
from jax.experimental import pallas as pl
from jax.experimental.pallas import tpu as pltpu


_EPS = 1e-12


def _body(x_ref, a_ref, o_ref, dots_ref, nsq_ref, coef_ref):
    xb = x_ref[...]
    a = a_ref[...]
    xnsq = jnp.sum(xb * xb, axis=1, keepdims=True)
    xinv = 1.0 / jnp.maximum(jnp.sqrt(xnsq), _EPS)
    xn = xb * xinv
    dots_ref[...] = jnp.sum(a * xn[None, :, :], axis=2)
    nsq_ref[...] = jnp.sum(a * a, axis=2)
    dots = dots_ref[...]
    nsq = nsq_ref[...]
    nainv = 1.0 / jnp.maximum(jnp.sqrt(nsq), _EPS)
    scores = dots * nainv
    m = jnp.max(scores, axis=0, keepdims=True)
    e = jnp.exp(scores - m)
    sinv = 1.0 / jnp.sum(e, axis=0, keepdims=True)
    coef_ref[...] = e * sinv * nainv
    coef = coef_ref[...]
    o_ref[...] = jnp.sum(a * coef[:, :, None], axis=0)


def kernel(x, attribute_feat):
    B, d = x.shape
    A = attribute_feat.shape[0]
    R = 256
    return pl.pallas_call(
        _body,
        grid=(B // R,),
        in_specs=[
            pl.BlockSpec((R, d), lambda i: (i, 0)),
            pl.BlockSpec((A, R, d), lambda i: (0, i, 0)),
        ],
        out_specs=pl.BlockSpec((R, d), lambda i: (i, 0)),
        out_shape=jax.ShapeDtypeStruct((B, d), jnp.float32),
        scratch_shapes=[
            pltpu.VMEM((A, R), jnp.float32),
            pltpu.VMEM((A, R), jnp.float32),
            pltpu.VMEM((A, R), jnp.float32),
        ],
    )(x, attribute_feat)
